# baseline (speedup 1.0000x reference)
import functools

import jax
import jax.numpy as jnp
from jax import lax
from jax.experimental import pallas as pl
from jax.experimental.pallas import tpu as pltpu
from jax.experimental.pallas import tpu_sc as plsc

N = 10000
E = 320000
F_IN = 128
H_DIM = 128
C_OUT = 10
K_KEEP = 5000

NPAD = 10240
NC = 2
NS = 16
NW = NC * NS
CH = 128
EPT = 10112
EP = NW * EPT
NCHUNK = EPT // CH
SL = NPAD // NS
DUMMY = N


def _mesh():
    return plsc.VectorSubcoreMesh(core_axis_name="c", subcore_axis_name="s")


@functools.cache
def _get_sc_deg1():
    @functools.partial(
        pl.kernel,
        out_type=jax.ShapeDtypeStruct((NW, NPAD), jnp.float32),
        mesh=_mesh(),
        scratch_types=[
            pltpu.VMEM((EPT,), jnp.int32),
            pltpu.VMEM((NPAD,), jnp.float32),
        ],
        compiler_params=pltpu.CompilerParams(needs_layout_passes=False),
    )
    def _sc_deg1(col_hbm, out_hbm, col_v, hist_v):
        cid = lax.axis_index("c")
        sid = lax.axis_index("s")
        wid = cid * NS + sid

        def zero(i, _):
            hist_v[pl.ds(i * 16, 16)] = jnp.zeros((16,), jnp.float32)
            return 0

        lax.fori_loop(0, NPAD // 16, zero, 0)
        pltpu.sync_copy(col_hbm.at[pl.ds(wid * EPT, EPT)], col_v)
        ones = jnp.full((16,), 1.0, jnp.float32)

        def body(j, _):
            idx = col_v[pl.ds(j * 16, 16)]
            plsc.addupdate_scatter(hist_v, [idx], ones)
            return 0

        lax.fori_loop(0, EPT // 16, body, 0)
        pltpu.sync_copy(hist_v, out_hbm.at[wid])

    return _sc_deg1


@functools.cache
def _get_sc_deg2():
    @functools.partial(
        pl.kernel,
        out_type=jax.ShapeDtypeStruct((NW, NPAD), jnp.float32),
        mesh=_mesh(),
        scratch_types=[
            pltpu.VMEM((EPT,), jnp.int32),
            pltpu.VMEM((EPT,), jnp.int32),
            pltpu.VMEM((NPAD,), jnp.float32),
            pltpu.VMEM((NPAD,), jnp.float32),
        ],
        compiler_params=pltpu.CompilerParams(needs_layout_passes=False),
    )
    def _sc_deg2(row_hbm, col_hbm, keep_hbm, out_hbm,
                 row_v, col_v, keep_v, hist_v):
        cid = lax.axis_index("c")
        sid = lax.axis_index("s")
        wid = cid * NS + sid

        def zero(i, _):
            hist_v[pl.ds(i * 16, 16)] = jnp.zeros((16,), jnp.float32)
            return 0

        lax.fori_loop(0, NPAD // 16, zero, 0)
        pltpu.sync_copy(keep_hbm, keep_v)
        pltpu.sync_copy(row_hbm.at[pl.ds(wid * EPT, EPT)], row_v)
        pltpu.sync_copy(col_hbm.at[pl.ds(wid * EPT, EPT)], col_v)

        def body(j, _):
            r = row_v[pl.ds(j * 16, 16)]
            c = col_v[pl.ds(j * 16, 16)]
            vals = plsc.load_gather(keep_v, [r])
            plsc.addupdate_scatter(hist_v, [c], vals)
            return 0

        lax.fori_loop(0, EPT // 16, body, 0)
        pltpu.sync_copy(hist_v, out_hbm.at[wid])

    return _sc_deg2


@functools.cache
def _get_sc_msg(width):
    ch = 32 if width == H_DIM else 64
    nb = 6 if width == H_DIM else 8
    nchunk = EPT // ch
    nround = (nchunk + nb - 1) // nb
    ragged = nchunk % nb != 0

    @functools.partial(
        pl.kernel,
        out_type=jax.ShapeDtypeStruct((NC, NPAD, width), jnp.float32),
        mesh=_mesh(),
        scratch_types=[
            pltpu.VMEM((nchunk, ch), jnp.int32),
            pltpu.VMEM((nchunk, ch), jnp.int32),
            pltpu.VMEM_SHARED((NPAD, width), jnp.float32),
        ] + [pltpu.VMEM((ch, width), jnp.float32)] * nb
          + [pltpu.SemaphoreType.DMA] * (2 * nb),
        compiler_params=pltpu.CompilerParams(use_tc_tiling_on_sc=False),
    )
    def _sc_msg(y_hbm, row_hbm, col_hbm, z_hbm, out_hbm,
                row_v, col_v, acc_s, *bufs_and_sems):
        rows = bufs_and_sems[:nb]
        sem_g = bufs_and_sems[nb:2 * nb]
        sem_s = bufs_and_sems[2 * nb:]
        cid = lax.axis_index("c")
        sid = lax.axis_index("s")
        wid = cid * NS + sid
        pltpu.sync_copy(row_hbm.at[wid], row_v)
        pltpu.sync_copy(col_hbm.at[wid], col_v)
        pltpu.sync_copy(z_hbm, acc_s.at[pl.ds(sid * SL, SL)])
        plsc.subcore_barrier()

        def start_gather(c, b):
            pltpu.async_copy(y_hbm.at[row_v.at[c]], rows[b], sem_g[b])

        def wait_gather(c, b):
            pltpu.make_async_copy(y_hbm.at[row_v.at[c]], rows[b],
                                  sem_g[b]).wait()

        def start_scat(c, b):
            pltpu.async_copy(rows[b], acc_s.at[col_v.at[c]], sem_s[b],
                             add=True)

        def wait_scat(c, b):
            pltpu.make_async_copy(rows[b], acc_s.at[col_v.at[c]],
                                  sem_s[b]).wait()

        def guarded(cond, fn):
            pl.when(cond)(fn)

        for b in range(nb):
            start_gather(b, b)

        def gather_to_scat(c, b):
            wait_gather(c, b)
            start_scat(c, b)

        def scat_to_gather(c2, b):
            wait_scat(c2 - nb, b)
            start_gather(c2, b)

        def round_(r, _):
            for b in range(nb):
                c = r * nb + b
                guarded(c < nchunk, functools.partial(gather_to_scat, c, b))
            for b in range(nb):
                c2 = (r + 1) * nb + b
                guarded(c2 < nchunk, functools.partial(scat_to_gather, c2, b))
            return 0

        lax.fori_loop(0, nround, round_, 0)
        for b in range(nb):
            c = ((nchunk - 1 - b) // nb) * nb + b
            wait_scat(c, b)
        plsc.subcore_barrier()
        pltpu.sync_copy(acc_s.at[pl.ds(sid * SL, SL)],
                        out_hbm.at[cid, pl.ds(sid * SL, SL)])

    return _sc_msg


def _tc1_body(x_ref, t_ref, w_ref, xw_ref):
    xw_ref[...] = lax.dot_general(
        x_ref[...] * t_ref[...], w_ref[...],
        (((1,), (0,)), ((), ())), precision=lax.Precision.HIGHEST)


def _tc2a_body(deg_ref, xw_ref, y_ref, dinv_ref):
    ones = jnp.ones((NW, 1), jnp.float32)
    deg = lax.dot_general(deg_ref[...], ones, (((0,), (0,)), ((), ())),
                          precision=lax.Precision.HIGHEST)
    dinv = lax.rsqrt(deg + 1.0)
    dinv_ref[...] = dinv
    y_ref[...] = dinv * xw_ref[...]


def _tc2_body(a_ref, xw_ref, dinv_ref, b1_ref, p_ref, w2_ref,
              h2w_ref, keep_ref):
    acc = a_ref[0] + a_ref[1]
    dinv = dinv_ref[...]
    h = dinv * acc + (dinv * dinv) * xw_ref[...] + b1_ref[...]
    p = p_ref[...]
    pn = lax.rsqrt(jnp.sum(p * p))
    sraw = lax.dot_general(h, p, (((1,), (0,)), ((), ())),
                           precision=lax.Precision.HIGHEST)
    s = jnp.tanh(sraw * pn)
    ikey = lax.bitcast_convert_type(s, jnp.int32)
    key = jnp.where(ikey < 0, ikey ^ jnp.int32(0x7FFFFFFF), ikey)
    valid = lax.broadcasted_iota(jnp.int32, (NPAD, 1), 0) < N
    kf = jnp.float32(K_KEEP)

    def bis(i, lh):
        lo, hi = lh
        mid = lo + lax.shift_right_arithmetic(hi - lo, 1)
        cnt = jnp.sum(jnp.where(valid & (key >= mid), 1.0, 0.0))
        big = cnt >= kf
        return (jnp.where(big, mid, lo), jnp.where(big, hi, mid))

    lo0 = jnp.int32(-1065353218)
    hi0 = jnp.int32(1065353217)
    lo, hi = lax.fori_loop(0, 34, bis, (lo0, hi0))
    keep = jnp.where(valid & (key >= lo), 1.0, 0.0)
    keep_ref[...] = keep
    hp = jnp.maximum(h * s, 0.0) * keep
    h2w_ref[...] = lax.dot_general(hp, w2_ref[...], (((1,), (0,)), ((), ())),
                                   precision=lax.Precision.HIGHEST)


def _tc3_body(deg_ref, h2w_ref, keep_ref, y2_ref, d2k_ref):
    ones = jnp.ones((NW, 1), jnp.float32)
    deg2 = lax.dot_general(deg_ref[...], ones, (((0,), (0,)), ((), ())),
                           precision=lax.Precision.HIGHEST)
    d2k = keep_ref[...] * lax.rsqrt(deg2 + 1.0)
    d2k_ref[...] = d2k
    y2_ref[...] = d2k * h2w_ref[...]


def _tc4_body(a_ref, h2w_ref, d2k_ref, keep_ref, b2_ref, wl_ref, bl_ref,
              out_ref):
    a2 = a_ref[0] + a_ref[1]
    d2k = d2k_ref[...]
    o = keep_ref[...] * jnp.maximum(
        d2k * a2 + (d2k * d2k) * h2w_ref[...] + b2_ref[...], 0.0)
    ones = jnp.ones((1, NPAD), jnp.float32)
    colsum = lax.dot_general(ones, o, (((1,), (0,)), ((), ())),
                             precision=lax.Precision.HIGHEST)
    mean = colsum * (1.0 / K_KEEP)
    out_ref[...] = lax.dot_general(mean, wl_ref[...], (((1,), (0,)), ((), ())),
                                   precision=lax.Precision.HIGHEST) + bl_ref[...]


def _tc(body, out_shape, *args):
    return pl.pallas_call(body, out_shape=out_shape)(*args)


@jax.jit
def kernel(x, t_x, edge_index, batch, W1, b1, p, W2, b2, Wl, bl):
    f32 = jnp.float32
    x = jnp.pad(x.astype(f32), ((0, NPAD - N), (0, 0)))
    t_x = jnp.pad(t_x.astype(f32), ((0, NPAD - N), (0, 0)))
    ei = edge_index.astype(jnp.int32)
    rowp = jnp.concatenate([ei[0], jnp.zeros((EP - E,), jnp.int32)])
    colp = jnp.concatenate([ei[1], jnp.full((EP - E,), DUMMY, jnp.int32)])
    row3a = rowp.reshape(NW, EPT // 32, 32)
    col3a = colp.reshape(NW, EPT // 32, 32)
    row3b = rowp.reshape(NW, EPT // 64, 64)
    col3b = colp.reshape(NW, EPT // 64, 64)
    b1r = b1.astype(f32).reshape(1, H_DIM)
    pc = p.astype(f32).reshape(H_DIM, 1)
    b2r = b2.astype(f32).reshape(1, 32)
    blr = bl.astype(f32).reshape(1, C_OUT)
    z128 = jnp.zeros((SL, H_DIM), f32)
    z32 = jnp.zeros((SL, 32), f32)

    xw = _tc(_tc1_body, jax.ShapeDtypeStruct((NPAD, H_DIM), f32),
             x, t_x, W1.astype(f32))
    deg1p = _get_sc_deg1()(colp)
    y1, dinv = _tc(_tc2a_body,
                   (jax.ShapeDtypeStruct((NPAD, H_DIM), f32),
                    jax.ShapeDtypeStruct((NPAD, 1), f32)),
                   deg1p, xw)
    acc1p = _get_sc_msg(H_DIM)(y1, row3a, col3a, z128)
    h2w, keep = _tc(_tc2_body,
                    (jax.ShapeDtypeStruct((NPAD, 32), f32),
                     jax.ShapeDtypeStruct((NPAD, 1), f32)),
                    acc1p, xw, dinv, b1r, pc, W2.astype(f32))
    deg2p = _get_sc_deg2()(rowp, colp, keep.reshape(NPAD))
    y2, d2k = _tc(_tc3_body,
                  (jax.ShapeDtypeStruct((NPAD, 32), f32),
                   jax.ShapeDtypeStruct((NPAD, 1), f32)),
                  deg2p, h2w, keep)
    acc2p = _get_sc_msg(32)(y2, row3b, col3b, z32)
    out = _tc(_tc4_body, jax.ShapeDtypeStruct((1, C_OUT), f32),
              acc2p, h2w, d2k, keep, b2r, Wl.astype(f32), blr)
    return out


# speedup vs baseline: 1.0025x; 1.0025x over previous
import functools

import jax
import jax.numpy as jnp
from jax import lax
from jax.experimental import pallas as pl
from jax.experimental.pallas import tpu as pltpu
from jax.experimental.pallas import tpu_sc as plsc

N = 10000
E = 320000
F_IN = 128
H_DIM = 128
C_OUT = 10
K_KEEP = 5000

NPAD = 10240
NC = 2
NS = 16
NW = NC * NS
CH = 128
EPT = 10112
EP = NW * EPT
NCHUNK = EPT // CH
SL = NPAD // NS
DUMMY = N


def _mesh():
    return plsc.VectorSubcoreMesh(core_axis_name="c", subcore_axis_name="s")


@functools.cache
def _get_sc_deg1():
    @functools.partial(
        pl.kernel,
        out_type=jax.ShapeDtypeStruct((NW, NPAD), jnp.float32),
        mesh=_mesh(),
        scratch_types=[
            pltpu.VMEM((EPT,), jnp.int32),
            pltpu.VMEM((NPAD,), jnp.float32),
        ],
        compiler_params=pltpu.CompilerParams(needs_layout_passes=False),
    )
    def _sc_deg1(col_hbm, out_hbm, col_v, hist_v):
        cid = lax.axis_index("c")
        sid = lax.axis_index("s")
        wid = cid * NS + sid

        def zero(i, _):
            hist_v[pl.ds(i * 16, 16)] = jnp.zeros((16,), jnp.float32)
            return 0

        lax.fori_loop(0, NPAD // 16, zero, 0)
        pltpu.sync_copy(col_hbm.at[pl.ds(wid * EPT, EPT)], col_v)
        ones = jnp.full((16,), 1.0, jnp.float32)

        def body(j, _):
            idx = col_v[pl.ds(j * 16, 16)]
            plsc.addupdate_scatter(hist_v, [idx], ones)
            return 0

        lax.fori_loop(0, EPT // 16, body, 0)
        pltpu.sync_copy(hist_v, out_hbm.at[wid])

    return _sc_deg1


@functools.cache
def _get_sc_deg2():
    @functools.partial(
        pl.kernel,
        out_type=jax.ShapeDtypeStruct((NW, NPAD), jnp.float32),
        mesh=_mesh(),
        scratch_types=[
            pltpu.VMEM((EPT,), jnp.int32),
            pltpu.VMEM((EPT,), jnp.int32),
            pltpu.VMEM((NPAD,), jnp.float32),
            pltpu.VMEM((NPAD,), jnp.float32),
        ],
        compiler_params=pltpu.CompilerParams(needs_layout_passes=False),
    )
    def _sc_deg2(row_hbm, col_hbm, keep_hbm, out_hbm,
                 row_v, col_v, keep_v, hist_v):
        cid = lax.axis_index("c")
        sid = lax.axis_index("s")
        wid = cid * NS + sid

        def zero(i, _):
            hist_v[pl.ds(i * 16, 16)] = jnp.zeros((16,), jnp.float32)
            return 0

        lax.fori_loop(0, NPAD // 16, zero, 0)
        pltpu.sync_copy(keep_hbm, keep_v)
        pltpu.sync_copy(row_hbm.at[pl.ds(wid * EPT, EPT)], row_v)
        pltpu.sync_copy(col_hbm.at[pl.ds(wid * EPT, EPT)], col_v)

        def body(j, _):
            r = row_v[pl.ds(j * 16, 16)]
            c = col_v[pl.ds(j * 16, 16)]
            vals = plsc.load_gather(keep_v, [r])
            plsc.addupdate_scatter(hist_v, [c], vals)
            return 0

        lax.fori_loop(0, EPT // 16, body, 0)
        pltpu.sync_copy(hist_v, out_hbm.at[wid])

    return _sc_deg2


@functools.cache
def _get_sc_msg(width):
    ch = 32 if width == H_DIM else CH
    nb = 6 if width == H_DIM else 8
    nchunk = EPT // ch
    nround = (nchunk + nb - 1) // nb
    ragged = nchunk % nb != 0

    @functools.partial(
        pl.kernel,
        out_type=jax.ShapeDtypeStruct((NC, NPAD, width), jnp.float32),
        mesh=_mesh(),
        scratch_types=[
            pltpu.VMEM((nchunk, ch), jnp.int32),
            pltpu.VMEM((nchunk, ch), jnp.int32),
            pltpu.VMEM_SHARED((NPAD, width), jnp.float32),
        ] + [pltpu.VMEM((ch, width), jnp.float32)] * nb
          + [pltpu.SemaphoreType.DMA] * (2 * nb),
        compiler_params=pltpu.CompilerParams(use_tc_tiling_on_sc=False),
    )
    def _sc_msg(y_hbm, row_hbm, col_hbm, z_hbm, out_hbm,
                row_v, col_v, acc_s, *bufs_and_sems):
        rows = bufs_and_sems[:nb]
        sem_g = bufs_and_sems[nb:2 * nb]
        sem_s = bufs_and_sems[2 * nb:]
        cid = lax.axis_index("c")
        sid = lax.axis_index("s")
        wid = cid * NS + sid
        pltpu.sync_copy(row_hbm.at[wid], row_v)
        pltpu.sync_copy(col_hbm.at[wid], col_v)
        pltpu.sync_copy(z_hbm, acc_s.at[pl.ds(sid * SL, SL)])
        plsc.subcore_barrier()

        def start_gather(c, b):
            pltpu.async_copy(y_hbm.at[row_v.at[c]], rows[b], sem_g[b])

        def wait_gather(c, b):
            pltpu.make_async_copy(y_hbm.at[row_v.at[c]], rows[b],
                                  sem_g[b]).wait()

        def start_scat(c, b):
            pltpu.async_copy(rows[b], acc_s.at[col_v.at[c]], sem_s[b],
                             add=True)

        def wait_scat(c, b):
            pltpu.make_async_copy(rows[b], acc_s.at[col_v.at[c]],
                                  sem_s[b]).wait()

        def guarded(cond, fn):
            pl.when(cond)(fn)

        for b in range(nb):
            start_gather(b, b)

        def gather_to_scat(c, b):
            wait_gather(c, b)
            start_scat(c, b)

        def scat_to_gather(c2, b):
            wait_scat(c2 - nb, b)
            start_gather(c2, b)

        def round_(r, _):
            for b in range(nb):
                c = r * nb + b
                guarded(c < nchunk, functools.partial(gather_to_scat, c, b))
            for b in range(nb):
                c2 = (r + 1) * nb + b
                guarded(c2 < nchunk, functools.partial(scat_to_gather, c2, b))
            return 0

        lax.fori_loop(0, nround, round_, 0)
        for b in range(nb):
            c = ((nchunk - 1 - b) // nb) * nb + b
            wait_scat(c, b)
        plsc.subcore_barrier()
        pltpu.sync_copy(acc_s.at[pl.ds(sid * SL, SL)],
                        out_hbm.at[cid, pl.ds(sid * SL, SL)])

    return _sc_msg


def _tc1_body(x_ref, t_ref, w_ref, xw_ref):
    xw_ref[...] = lax.dot_general(
        x_ref[...] * t_ref[...], w_ref[...],
        (((1,), (0,)), ((), ())), precision=lax.Precision.HIGHEST)


def _tc2a_body(deg_ref, xw_ref, y_ref, dinv_ref):
    ones = jnp.ones((NW, 1), jnp.float32)
    deg = lax.dot_general(deg_ref[...], ones, (((0,), (0,)), ((), ())),
                          precision=lax.Precision.HIGHEST)
    dinv = lax.rsqrt(deg + 1.0)
    dinv_ref[...] = dinv
    y_ref[...] = dinv * xw_ref[...]


def _tc2_body(a_ref, xw_ref, dinv_ref, b1_ref, p_ref, w2_ref,
              h2w_ref, keep_ref):
    acc = a_ref[0] + a_ref[1]
    dinv = dinv_ref[...]
    h = dinv * acc + (dinv * dinv) * xw_ref[...] + b1_ref[...]
    p = p_ref[...]
    pn = lax.rsqrt(jnp.sum(p * p))
    sraw = lax.dot_general(h, p, (((1,), (0,)), ((), ())),
                           precision=lax.Precision.HIGHEST)
    s = jnp.tanh(sraw * pn)
    ikey = lax.bitcast_convert_type(s, jnp.int32)
    key = jnp.where(ikey < 0, ikey ^ jnp.int32(0x7FFFFFFF), ikey)
    valid = lax.broadcasted_iota(jnp.int32, (NPAD, 1), 0) < N
    kf = jnp.float32(K_KEEP)

    def bis(i, lh):
        lo, hi = lh
        mid = lo + lax.shift_right_arithmetic(hi - lo, 1)
        cnt = jnp.sum(jnp.where(valid & (key >= mid), 1.0, 0.0))
        big = cnt >= kf
        return (jnp.where(big, mid, lo), jnp.where(big, hi, mid))

    lo0 = jnp.int32(-1065353218)
    hi0 = jnp.int32(1065353217)
    lo, hi = lax.fori_loop(0, 34, bis, (lo0, hi0))
    keep = jnp.where(valid & (key >= lo), 1.0, 0.0)
    keep_ref[...] = keep
    hp = jnp.maximum(h * s, 0.0) * keep
    h2w_ref[...] = lax.dot_general(hp, w2_ref[...], (((1,), (0,)), ((), ())),
                                   precision=lax.Precision.HIGHEST)


def _tc3_body(deg_ref, h2w_ref, keep_ref, y2_ref, d2k_ref):
    ones = jnp.ones((NW, 1), jnp.float32)
    deg2 = lax.dot_general(deg_ref[...], ones, (((0,), (0,)), ((), ())),
                           precision=lax.Precision.HIGHEST)
    d2k = keep_ref[...] * lax.rsqrt(deg2 + 1.0)
    d2k_ref[...] = d2k
    y2_ref[...] = d2k * h2w_ref[...]


def _tc4_body(a_ref, h2w_ref, d2k_ref, keep_ref, b2_ref, wl_ref, bl_ref,
              out_ref):
    a2 = a_ref[0] + a_ref[1]
    d2k = d2k_ref[...]
    o = keep_ref[...] * jnp.maximum(
        d2k * a2 + (d2k * d2k) * h2w_ref[...] + b2_ref[...], 0.0)
    ones = jnp.ones((1, NPAD), jnp.float32)
    colsum = lax.dot_general(ones, o, (((1,), (0,)), ((), ())),
                             precision=lax.Precision.HIGHEST)
    mean = colsum * (1.0 / K_KEEP)
    out_ref[...] = lax.dot_general(mean, wl_ref[...], (((1,), (0,)), ((), ())),
                                   precision=lax.Precision.HIGHEST) + bl_ref[...]


def _tc(body, out_shape, *args):
    return pl.pallas_call(body, out_shape=out_shape)(*args)


@jax.jit
def kernel(x, t_x, edge_index, batch, W1, b1, p, W2, b2, Wl, bl):
    f32 = jnp.float32
    x = jnp.pad(x.astype(f32), ((0, NPAD - N), (0, 0)))
    t_x = jnp.pad(t_x.astype(f32), ((0, NPAD - N), (0, 0)))
    ei = edge_index.astype(jnp.int32)
    rowp = jnp.concatenate([ei[0], jnp.zeros((EP - E,), jnp.int32)])
    colp = jnp.concatenate([ei[1], jnp.full((EP - E,), DUMMY, jnp.int32)])
    row3a = rowp.reshape(NW, EPT // 32, 32)
    col3a = colp.reshape(NW, EPT // 32, 32)
    row3b = rowp.reshape(NW, NCHUNK, CH)
    col3b = colp.reshape(NW, NCHUNK, CH)
    b1r = b1.astype(f32).reshape(1, H_DIM)
    pc = p.astype(f32).reshape(H_DIM, 1)
    b2r = b2.astype(f32).reshape(1, 32)
    blr = bl.astype(f32).reshape(1, C_OUT)
    z128 = jnp.zeros((SL, H_DIM), f32)
    z32 = jnp.zeros((SL, 32), f32)

    xw = _tc(_tc1_body, jax.ShapeDtypeStruct((NPAD, H_DIM), f32),
             x, t_x, W1.astype(f32))
    deg1p = _get_sc_deg1()(colp)
    y1, dinv = _tc(_tc2a_body,
                   (jax.ShapeDtypeStruct((NPAD, H_DIM), f32),
                    jax.ShapeDtypeStruct((NPAD, 1), f32)),
                   deg1p, xw)
    acc1p = _get_sc_msg(H_DIM)(y1, row3a, col3a, z128)
    h2w, keep = _tc(_tc2_body,
                    (jax.ShapeDtypeStruct((NPAD, 32), f32),
                     jax.ShapeDtypeStruct((NPAD, 1), f32)),
                    acc1p, xw, dinv, b1r, pc, W2.astype(f32))
    deg2p = _get_sc_deg2()(rowp, colp, keep.reshape(NPAD))
    y2, d2k = _tc(_tc3_body,
                  (jax.ShapeDtypeStruct((NPAD, 32), f32),
                   jax.ShapeDtypeStruct((NPAD, 1), f32)),
                  deg2p, h2w, keep)
    acc2p = _get_sc_msg(32)(y2, row3b, col3b, z32)
    out = _tc(_tc4_body, jax.ShapeDtypeStruct((1, C_OUT), f32),
              acc2p, h2w, d2k, keep, b2r, Wl.astype(f32), blr)
    return out


# speedup vs baseline: 1.0030x; 1.0005x over previous
import functools

import jax
import jax.numpy as jnp
from jax import lax
from jax.experimental import pallas as pl
from jax.experimental.pallas import tpu as pltpu
from jax.experimental.pallas import tpu_sc as plsc

N = 10000
E = 320000
F_IN = 128
H_DIM = 128
C_OUT = 10
K_KEEP = 5000

NPAD = 10240
NC = 2
NS = 16
NW = NC * NS
CH = 128
EPT = 10112
EP = NW * EPT
NCHUNK = EPT // CH
SL = NPAD // NS
DUMMY = N


def _mesh():
    return plsc.VectorSubcoreMesh(core_axis_name="c", subcore_axis_name="s")


@functools.cache
def _get_sc_deg1():
    @functools.partial(
        pl.kernel,
        out_type=jax.ShapeDtypeStruct((NW, NPAD), jnp.float32),
        mesh=_mesh(),
        scratch_types=[
            pltpu.VMEM((EPT,), jnp.int32),
            pltpu.VMEM((NPAD,), jnp.float32),
        ],
        compiler_params=pltpu.CompilerParams(needs_layout_passes=False),
    )
    def _sc_deg1(col_hbm, out_hbm, col_v, hist_v):
        cid = lax.axis_index("c")
        sid = lax.axis_index("s")
        wid = cid * NS + sid

        def zero(i, _):
            hist_v[pl.ds(i * 16, 16)] = jnp.zeros((16,), jnp.float32)
            return 0

        lax.fori_loop(0, NPAD // 16, zero, 0)
        pltpu.sync_copy(col_hbm.at[pl.ds(wid * EPT, EPT)], col_v)
        ones = jnp.full((16,), 1.0, jnp.float32)

        def body(j, _):
            idx = col_v[pl.ds(j * 16, 16)]
            plsc.addupdate_scatter(hist_v, [idx], ones)
            return 0

        lax.fori_loop(0, EPT // 16, body, 0)
        pltpu.sync_copy(hist_v, out_hbm.at[wid])

    return _sc_deg1


@functools.cache
def _get_sc_deg2():
    @functools.partial(
        pl.kernel,
        out_type=jax.ShapeDtypeStruct((NW, NPAD), jnp.float32),
        mesh=_mesh(),
        scratch_types=[
            pltpu.VMEM((EPT,), jnp.int32),
            pltpu.VMEM((EPT,), jnp.int32),
            pltpu.VMEM((NPAD,), jnp.float32),
            pltpu.VMEM((NPAD,), jnp.float32),
        ],
        compiler_params=pltpu.CompilerParams(needs_layout_passes=False),
    )
    def _sc_deg2(row_hbm, col_hbm, keep_hbm, out_hbm,
                 row_v, col_v, keep_v, hist_v):
        cid = lax.axis_index("c")
        sid = lax.axis_index("s")
        wid = cid * NS + sid

        def zero(i, _):
            hist_v[pl.ds(i * 16, 16)] = jnp.zeros((16,), jnp.float32)
            return 0

        lax.fori_loop(0, NPAD // 16, zero, 0)
        pltpu.sync_copy(keep_hbm, keep_v)
        pltpu.sync_copy(row_hbm.at[pl.ds(wid * EPT, EPT)], row_v)
        pltpu.sync_copy(col_hbm.at[pl.ds(wid * EPT, EPT)], col_v)

        def body(j, _):
            r = row_v[pl.ds(j * 16, 16)]
            c = col_v[pl.ds(j * 16, 16)]
            vals = plsc.load_gather(keep_v, [r])
            plsc.addupdate_scatter(hist_v, [c], vals)
            return 0

        lax.fori_loop(0, EPT // 16, body, 0)
        pltpu.sync_copy(hist_v, out_hbm.at[wid])

    return _sc_deg2


@functools.cache
def _get_sc_msg(width):
    ch = 32 if width == H_DIM else CH
    nb = 6 if width == H_DIM else 8
    nchunk = EPT // ch
    nround = (nchunk + nb - 1) // nb

    @functools.partial(
        pl.kernel,
        out_type=jax.ShapeDtypeStruct((NC, NPAD, width), jnp.float32),
        mesh=_mesh(),
        scratch_types=[
            pltpu.VMEM((nchunk, ch), jnp.int32),
            pltpu.VMEM((nchunk, ch), jnp.int32),
            pltpu.VMEM_SHARED((NPAD, width), jnp.float32),
        ] + [pltpu.VMEM((ch, width), jnp.float32)] * nb
          + [pltpu.SemaphoreType.DMA] * (2 * nb),
        compiler_params=pltpu.CompilerParams(use_tc_tiling_on_sc=False),
    )
    def _sc_msg(y_hbm, row_hbm, col_hbm, z_hbm, out_hbm,
                row_v, col_v, acc_s, *bufs_and_sems):
        rows = bufs_and_sems[:nb]
        sem_g = bufs_and_sems[nb:2 * nb]
        sem_s = bufs_and_sems[2 * nb:]
        cid = lax.axis_index("c")
        sid = lax.axis_index("s")
        wid = cid * NS + sid
        pltpu.sync_copy(row_hbm.at[wid], row_v)
        pltpu.sync_copy(col_hbm.at[wid], col_v)
        pltpu.sync_copy(z_hbm, acc_s.at[pl.ds(sid * SL, SL)])
        plsc.subcore_barrier()

        def start_gather(c, b):
            pltpu.async_copy(y_hbm.at[row_v.at[c]], rows[b], sem_g[b])

        def wait_gather(c, b):
            pltpu.make_async_copy(y_hbm.at[row_v.at[c]], rows[b],
                                  sem_g[b]).wait()

        def start_scat(c, b):
            pltpu.async_copy(rows[b], acc_s.at[col_v.at[c]], sem_s[b],
                             add=True)

        def wait_scat(c, b):
            pltpu.make_async_copy(rows[b], acc_s.at[col_v.at[c]],
                                  sem_s[b]).wait()

        def guarded(cond, fn):
            pl.when(cond)(fn)

        for b in range(nb):
            start_gather(b, b)

        def gather_to_scat(c, b):
            wait_gather(c, b)
            start_scat(c, b)

        def scat_to_gather(c2, b):
            wait_scat(c2 - nb, b)
            start_gather(c2, b)

        def round_(r, _):
            for b in range(nb):
                c = r * nb + b
                guarded(c < nchunk, functools.partial(gather_to_scat, c, b))
            for b in range(nb):
                c2 = (r + 1) * nb + b
                guarded(c2 < nchunk, functools.partial(scat_to_gather, c2, b))
            return 0

        lax.fori_loop(0, nround, round_, 0)
        for b in range(nb):
            c = ((nchunk - 1 - b) // nb) * nb + b
            wait_scat(c, b)
        plsc.subcore_barrier()
        pltpu.sync_copy(acc_s.at[pl.ds(sid * SL, SL)],
                        out_hbm.at[cid, pl.ds(sid * SL, SL)])

    return _sc_msg


def _tc1_body(x_ref, t_ref, w_ref, xw_ref):
    xw_ref[...] = lax.dot_general(
        x_ref[...] * t_ref[...], w_ref[...],
        (((1,), (0,)), ((), ())), precision=lax.Precision.HIGHEST)


def _tc2a_body(deg_ref, xw_ref, y_ref, dinv_ref):
    ones = jnp.ones((NW, 1), jnp.float32)
    deg = lax.dot_general(deg_ref[...], ones, (((0,), (0,)), ((), ())),
                          precision=lax.Precision.HIGHEST)
    dinv = lax.rsqrt(deg + 1.0)
    dinv_ref[...] = dinv
    y_ref[...] = dinv * xw_ref[...]


def _tc2_body(a_ref, xw_ref, dinv_ref, b1_ref, p_ref, w2_ref,
              h2w_ref, keep_ref):
    acc = a_ref[0] + a_ref[1]
    dinv = dinv_ref[...]
    h = dinv * acc + (dinv * dinv) * xw_ref[...] + b1_ref[...]
    p = p_ref[...]
    pn = lax.rsqrt(jnp.sum(p * p))
    sraw = lax.dot_general(h, p, (((1,), (0,)), ((), ())),
                           precision=lax.Precision.HIGHEST)
    s = jnp.tanh(sraw * pn)
    ikey = lax.bitcast_convert_type(s, jnp.int32)
    key = jnp.where(ikey < 0, ikey ^ jnp.int32(0x7FFFFFFF), ikey)
    valid = lax.broadcasted_iota(jnp.int32, (NPAD, 1), 0) < N
    kf = jnp.float32(K_KEEP)

    def bis(i, lh):
        lo, hi = lh
        mid = lo + lax.shift_right_arithmetic(hi - lo, 1)
        cnt = jnp.sum(jnp.where(valid & (key >= mid), 1.0, 0.0))
        big = cnt >= kf
        return (jnp.where(big, mid, lo), jnp.where(big, hi, mid))

    lo0 = jnp.int32(-1065353218)
    hi0 = jnp.int32(1065353217)
    lo, hi = lax.fori_loop(0, 34, bis, (lo0, hi0))
    keep = jnp.where(valid & (key >= lo), 1.0, 0.0)
    keep_ref[...] = keep
    hp = jnp.maximum(h * s, 0.0) * keep
    h2w_ref[...] = lax.dot_general(hp, w2_ref[...], (((1,), (0,)), ((), ())),
                                   precision=lax.Precision.HIGHEST)


def _tc3_body(deg_ref, h2w_ref, keep_ref, y2_ref, d2k_ref):
    ones = jnp.ones((NW, 1), jnp.float32)
    deg2 = lax.dot_general(deg_ref[...], ones, (((0,), (0,)), ((), ())),
                           precision=lax.Precision.HIGHEST)
    d2k = keep_ref[...] * lax.rsqrt(deg2 + 1.0)
    d2k_ref[...] = d2k
    y2_ref[...] = d2k * h2w_ref[...]


def _tc4_body(a_ref, h2w_ref, d2k_ref, keep_ref, b2_ref, wl_ref, bl_ref,
              out_ref):
    a2 = a_ref[0] + a_ref[1]
    d2k = d2k_ref[...]
    o = keep_ref[...] * jnp.maximum(
        d2k * a2 + (d2k * d2k) * h2w_ref[...] + b2_ref[...], 0.0)
    ones = jnp.ones((1, NPAD), jnp.float32)
    colsum = lax.dot_general(ones, o, (((1,), (0,)), ((), ())),
                             precision=lax.Precision.HIGHEST)
    mean = colsum * (1.0 / K_KEEP)
    out_ref[...] = lax.dot_general(mean, wl_ref[...], (((1,), (0,)), ((), ())),
                                   precision=lax.Precision.HIGHEST) + bl_ref[...]


def _tc(body, out_shape, *args):
    return pl.pallas_call(body, out_shape=out_shape)(*args)


@jax.jit
def kernel(x, t_x, edge_index, batch, W1, b1, p, W2, b2, Wl, bl):
    f32 = jnp.float32
    x = jnp.pad(x.astype(f32), ((0, NPAD - N), (0, 0)))
    t_x = jnp.pad(t_x.astype(f32), ((0, NPAD - N), (0, 0)))
    ei = edge_index.astype(jnp.int32)
    rowp = jnp.concatenate([ei[0], jnp.zeros((EP - E,), jnp.int32)])
    colp = jnp.concatenate([ei[1], jnp.full((EP - E,), DUMMY, jnp.int32)])
    row3a = rowp.reshape(NW, EPT // 32, 32)
    col3a = colp.reshape(NW, EPT // 32, 32)
    row3b = rowp.reshape(NW, NCHUNK, CH)
    col3b = colp.reshape(NW, NCHUNK, CH)
    b1r = b1.astype(f32).reshape(1, H_DIM)
    pc = p.astype(f32).reshape(H_DIM, 1)
    b2r = b2.astype(f32).reshape(1, 32)
    blr = bl.astype(f32).reshape(1, C_OUT)
    z128 = jnp.zeros((SL, H_DIM), f32)
    z32 = jnp.zeros((SL, 32), f32)

    xw = _tc(_tc1_body, jax.ShapeDtypeStruct((NPAD, H_DIM), f32),
             x, t_x, W1.astype(f32))
    deg1p = _get_sc_deg1()(colp)
    y1, dinv = _tc(_tc2a_body,
                   (jax.ShapeDtypeStruct((NPAD, H_DIM), f32),
                    jax.ShapeDtypeStruct((NPAD, 1), f32)),
                   deg1p, xw)
    acc1p = _get_sc_msg(H_DIM)(y1, row3a, col3a, z128)
    h2w, keep = _tc(_tc2_body,
                    (jax.ShapeDtypeStruct((NPAD, 32), f32),
                     jax.ShapeDtypeStruct((NPAD, 1), f32)),
                    acc1p, xw, dinv, b1r, pc, W2.astype(f32))
    deg2p = _get_sc_deg2()(rowp, colp, keep.reshape(NPAD))
    y2, d2k = _tc(_tc3_body,
                  (jax.ShapeDtypeStruct((NPAD, 32), f32),
                   jax.ShapeDtypeStruct((NPAD, 1), f32)),
                  deg2p, h2w, keep)
    acc2p = _get_sc_msg(32)(y2, row3b, col3b, z32)
    out = _tc(_tc4_body, jax.ShapeDtypeStruct((1, C_OUT), f32),
              acc2p, h2w, d2k, keep, b2r, Wl.astype(f32), blr)
    return out
